# Initial kernel scaffold; baseline (speedup 1.0000x reference)
import jax
import jax.numpy as jnp
from jax.experimental import pallas as pl


def kernel(data_x, data_adj, W1, b1, prelu_a, W2, b2):
    raise NotImplementedError("write your pallas kernel here")


# speedup vs baseline: 15.8576x; 15.8576x over previous
import functools

import jax
import jax.numpy as jnp
from jax import lax
from jax.experimental import pallas as pl
from jax.experimental.pallas import tpu as pltpu
from jax.experimental.pallas import tpu_sc as plsc

N = 50000
E = 800000
HID = 64
NPAD = 50176
NROW, NLANE = 392, 128
NC, NS, L = 2, 16, 16
NW = NC * NS
B = 128
NB = E // B
RPT = NPAD // NS

_mesh = plsc.VectorSubcoreMesh(core_axis_name="c", subcore_axis_name="s")
f32 = jnp.float32
i32 = jnp.int32


def _fill(ref, n, value):
    def body(i, _):
        ref[pl.ds(i * L, L)] = jnp.full((L,), value, f32)
        return 0
    lax.fori_loop(0, n // L, body, 0)


def _hist_body(dst_hbm, out_hbm, idx_b, ones_v, zer_v, hist_sh):
    cid = lax.axis_index("c")
    sid = lax.axis_index("s")
    w = sid * NC + cid
    _fill(ones_v, B, 1.0)
    _fill(zer_v, RPT, 0.0)
    pltpu.sync_copy(zer_v, hist_sh.at[pl.ds(sid * RPT, RPT)])
    plsc.subcore_barrier()
    nb = (NB - w + NW - 1) // NW

    def body(j, _):
        off = (w + NW * j) * B
        pltpu.sync_copy(dst_hbm.at[pl.ds(off, B)], idx_b)
        pltpu.sync_copy(ones_v, hist_sh.at[idx_b], add=True)
        return 0
    lax.fori_loop(0, nb, body, 0)
    plsc.subcore_barrier()
    pltpu.sync_copy(hist_sh.at[pl.ds(sid * RPT, RPT)], zer_v)
    pltpu.sync_copy(zer_v, out_hbm.at[pl.ds(cid * NPAD + sid * RPT, RPT)])


_hist_call = pl.kernel(
    _hist_body,
    out_type=jax.ShapeDtypeStruct((NC * NPAD,), f32),
    mesh=_mesh,
    scratch_types=[
        pltpu.VMEM((B,), i32),
        pltpu.VMEM((B,), f32),
        pltpu.VMEM((RPT,), f32),
        pltpu.VMEM_SHARED((NPAD,), f32),
    ],
)


def _t_body(src_hbm, dst_hbm, u_hbm, out_hbm,
            idx_s, idx_d, val_b, zer_v, t_sh):
    cid = lax.axis_index("c")
    sid = lax.axis_index("s")
    w = sid * NC + cid
    _fill(zer_v, RPT, 0.0)
    pltpu.sync_copy(zer_v, t_sh.at[pl.ds(sid * RPT, RPT)])
    plsc.subcore_barrier()
    nb = (NB - w + NW - 1) // NW

    def body(j, _):
        off = (w + NW * j) * B
        pltpu.sync_copy(src_hbm.at[pl.ds(off, B)], idx_s)
        pltpu.sync_copy(dst_hbm.at[pl.ds(off, B)], idx_d)
        pltpu.sync_copy(u_hbm.at[idx_s], val_b)
        pltpu.sync_copy(val_b, t_sh.at[idx_d], add=True)
        return 0
    lax.fori_loop(0, nb, body, 0)
    plsc.subcore_barrier()
    pltpu.sync_copy(t_sh.at[pl.ds(sid * RPT, RPT)], zer_v)
    pltpu.sync_copy(zer_v, out_hbm.at[pl.ds(cid * NPAD + sid * RPT, RPT)])


_t_call = pl.kernel(
    _t_body,
    out_type=jax.ShapeDtypeStruct((NC * NPAD,), f32),
    mesh=_mesh,
    scratch_types=[
        pltpu.VMEM((B,), i32),
        pltpu.VMEM((B,), i32),
        pltpu.VMEM((B,), f32),
        pltpu.VMEM((RPT,), f32),
        pltpu.VMEM_SHARED((NPAD,), f32),
    ],
)


def _row_body(src_hbm, dst_hbm, ylo_hbm, yhi_hbm, out_hbm,
              idx_s, idx_d, stage, zrow_v, a_sh):
    cid = lax.axis_index("c")
    sid = lax.axis_index("s")
    ZR = 64

    def zinit(i, _):
        zrow_v[i, pl.ds(0, L)] = jnp.zeros((L,), f32)
        zrow_v[i, pl.ds(L, L)] = jnp.zeros((L,), f32)
        return 0
    lax.fori_loop(0, ZR, zinit, 0)
    for k in range(RPT // ZR):
        pltpu.sync_copy(zrow_v, a_sh.at[pl.ds(sid * RPT + k * ZR, ZR)])
    plsc.subcore_barrier()

    nb = (NB - sid + NS - 1) // NS

    def body(j, _):
        off = (sid + NS * j) * B
        pltpu.sync_copy(src_hbm.at[pl.ds(off, B)], idx_s)
        pltpu.sync_copy(dst_hbm.at[pl.ds(off, B)], idx_d)

        @pl.when(cid == 0)
        def _():
            pltpu.sync_copy(ylo_hbm.at[idx_s], stage)

        @pl.when(cid == 1)
        def _():
            pltpu.sync_copy(yhi_hbm.at[idx_s], stage)
        pltpu.sync_copy(stage, a_sh.at[idx_d], add=True)
        return 0
    lax.fori_loop(0, nb, body, 0)
    plsc.subcore_barrier()
    for k in range(RPT // ZR):
        pltpu.sync_copy(a_sh.at[pl.ds(sid * RPT + k * ZR, ZR)], zrow_v)
        pltpu.sync_copy(zrow_v,
                        out_hbm.at[pl.ds(cid * NPAD + sid * RPT + k * ZR, ZR)])


_row_call = pl.kernel(
    _row_body,
    out_type=jax.ShapeDtypeStruct((NC * NPAD, HID // 2), f32),
    mesh=_mesh,
    compiler_params=pltpu.CompilerParams(use_tc_tiling_on_sc=False),
    scratch_types=[
        pltpu.VMEM((B,), i32),
        pltpu.VMEM((B,), i32),
        pltpu.VMEM((B, HID // 2), f32),
        pltpu.VMEM((64, HID // 2), f32),
        pltpu.VMEM_SHARED((NPAD, HID // 2), f32),
    ],
)


def _tc1_body(hist_ref, x_ref, d_ref, u_ref):
    deg = hist_ref[0] + hist_ref[1] + 1.0
    d = lax.rsqrt(deg)
    d_ref[...] = d
    u_ref[...] = d * x_ref[...]


_tc1 = pl.pallas_call(
    _tc1_body,
    grid=(NROW // 8,),
    in_specs=[
        pl.BlockSpec((NC, 8, NLANE), lambda i: (0, i, 0)),
        pl.BlockSpec((8, NLANE), lambda i: (i, 0)),
    ],
    out_specs=[
        pl.BlockSpec((8, NLANE), lambda i: (i, 0)),
        pl.BlockSpec((8, NLANE), lambda i: (i, 0)),
    ],
    out_shape=[
        jax.ShapeDtypeStruct((NROW, NLANE), f32),
        jax.ShapeDtypeStruct((NROW, NLANE), f32),
    ],
)


def _tc2_body(t_ref, d_ref, x_ref, s_ref):
    d = d_ref[...]
    s_ref[...] = d * (t_ref[0] + t_ref[1] + d * x_ref[...])


_tc2 = pl.pallas_call(
    _tc2_body,
    grid=(NROW // 8,),
    in_specs=[
        pl.BlockSpec((NC, 8, NLANE), lambda i: (0, i, 0)),
        pl.BlockSpec((8, NLANE), lambda i: (i, 0)),
        pl.BlockSpec((8, NLANE), lambda i: (i, 0)),
    ],
    out_specs=pl.BlockSpec((8, NLANE), lambda i: (i, 0)),
    out_shape=jax.ShapeDtypeStruct((NROW, NLANE), f32),
)

RB = 1024


def _tc3_body(s_ref, d_ref, w1_ref, b1_ref, pa_ref, w2_ref,
              z_ref, ylo_ref, yhi_ref):
    s = s_ref[...]
    h = s * w1_ref[...] + b1_ref[...]
    a = pa_ref[0, 0]
    h = jnp.where(h >= 0, h, a * h)
    z = jnp.dot(h, w2_ref[...], precision=lax.Precision.HIGHEST,
                preferred_element_type=f32)
    y = d_ref[...] * z
    z_ref[...] = z
    ylo_ref[...] = y[:, :HID // 2]
    yhi_ref[...] = y[:, HID // 2:]


_tc3 = pl.pallas_call(
    _tc3_body,
    grid=(NPAD // RB,),
    in_specs=[
        pl.BlockSpec((RB, 1), lambda i: (i, 0)),
        pl.BlockSpec((RB, 1), lambda i: (i, 0)),
        pl.BlockSpec((1, HID), lambda i: (0, 0)),
        pl.BlockSpec((1, HID), lambda i: (0, 0)),
        pl.BlockSpec(memory_space=pltpu.SMEM),
        pl.BlockSpec((HID, HID), lambda i: (0, 0)),
    ],
    out_specs=[
        pl.BlockSpec((RB, HID), lambda i: (i, 0)),
        pl.BlockSpec((RB, HID // 2), lambda i: (i, 0)),
        pl.BlockSpec((RB, HID // 2), lambda i: (i, 0)),
    ],
    out_shape=[
        jax.ShapeDtypeStruct((NPAD, HID), f32),
        jax.ShapeDtypeStruct((NPAD, HID // 2), f32),
        jax.ShapeDtypeStruct((NPAD, HID // 2), f32),
    ],
)


def _tc4_body(alo_ref, ahi_ref, z_ref, d_ref, b2_ref, out_ref):
    d = d_ref[...]
    z = z_ref[...]
    b2 = b2_ref[...]
    lo = d * (alo_ref[...] + d * z[:, :HID // 2]) + b2[:, :HID // 2]
    hi = d * (ahi_ref[...] + d * z[:, HID // 2:]) + b2[:, HID // 2:]
    out_ref[...] = jnp.concatenate([lo, hi], axis=1)


_tc4 = pl.pallas_call(
    _tc4_body,
    grid=(NPAD // RB,),
    in_specs=[
        pl.BlockSpec((RB, HID // 2), lambda i: (i, 0)),
        pl.BlockSpec((RB, HID // 2), lambda i: (i, 0)),
        pl.BlockSpec((RB, HID), lambda i: (i, 0)),
        pl.BlockSpec((RB, 1), lambda i: (i, 0)),
        pl.BlockSpec((1, HID), lambda i: (0, 0)),
    ],
    out_specs=pl.BlockSpec((RB, HID), lambda i: (i, 0)),
    out_shape=jax.ShapeDtypeStruct((NPAD, HID), f32),
)


def kernel(data_x, data_adj, W1, b1, prelu_a, W2, b2):
    x = data_x[:, 0].astype(f32)
    xp = jnp.pad(x, (0, NPAD - N))
    src = data_adj[0].astype(i32)
    dst = data_adj[1].astype(i32)

    hist = _hist_call(dst)
    hist2 = hist.reshape(NC, NROW, NLANE)
    x2 = xp.reshape(NROW, NLANE)
    d2, u2 = _tc1(hist2, x2)

    tpart = _t_call(src, dst, u2.reshape(NPAD))
    s2 = _tc2(tpart.reshape(NC, NROW, NLANE), d2, x2)

    s_col = s2.reshape(NPAD, 1)
    d_col = d2.reshape(NPAD, 1)
    z, ylo, yhi = _tc3(s_col, d_col, W1.reshape(1, HID).astype(f32),
                       b1.reshape(1, HID).astype(f32),
                       prelu_a.reshape(1, 1).astype(f32), W2.astype(f32))

    apart = _row_call(src, dst, ylo, yhi)
    outp = _tc4(apart[:NPAD], apart[NPAD:], z, d_col,
                b2.reshape(1, HID).astype(f32))
    return outp[:N]


# speedup vs baseline: 24.1452x; 1.5226x over previous
import functools

import jax
import jax.numpy as jnp
from jax import lax
from jax.experimental import pallas as pl
from jax.experimental.pallas import tpu as pltpu
from jax.experimental.pallas import tpu_sc as plsc

N = 50000
E = 800000
HID = 64
NPAD = 50176
NROW, NLANE = 392, 128
NC, NS, L = 2, 16, 16
NW = NC * NS
B = 128
KG = 8
GBR = 400
CQR = 4
E2 = 819200
NCHR = E2 // (CQR * GBR * NS)
NGT = E2 // (KG * B * NW)
RPT = NPAD // NS

_mesh = plsc.VectorSubcoreMesh(core_axis_name="c", subcore_axis_name="s")
f32 = jnp.float32
i32 = jnp.int32


def _fill(ref, n, value):
    def body(i, _):
        ref[pl.ds(i * L, L)] = jnp.full((L,), value, f32)
        return 0
    lax.fori_loop(0, n // L, body, 0)


def _hist_body(dstg_hbm, out_hbm, idx_g, ones_v, zer_v, hist_sh):
    cid = lax.axis_index("c")
    sid = lax.axis_index("s")
    w = sid * NC + cid

    _fill(ones_v, KG * B, 1.0)
    _fill(zer_v, RPT, 0.0)
    pltpu.sync_copy(zer_v, hist_sh.at[pl.ds(sid * RPT, RPT)])
    plsc.subcore_barrier()

    def body(j, _):
        g = w * NGT + j
        pltpu.sync_copy(dstg_hbm.at[g], idx_g)
        pltpu.sync_copy(ones_v, hist_sh.at[idx_g], add=True)
        return 0
    lax.fori_loop(0, NGT, body, 0)
    plsc.subcore_barrier()
    pltpu.sync_copy(hist_sh.at[pl.ds(sid * RPT, RPT)], zer_v)
    pltpu.sync_copy(zer_v, out_hbm.at[pl.ds(cid * NPAD + sid * RPT, RPT)])


_hist_call = pl.kernel(
    _hist_body,
    out_type=jax.ShapeDtypeStruct((NC * NPAD,), f32),
    mesh=_mesh,
    compiler_params=pltpu.CompilerParams(use_tc_tiling_on_sc=False),
    scratch_types=[
        pltpu.VMEM((KG * B,), i32),
        pltpu.VMEM((KG * B,), f32),
        pltpu.VMEM((RPT,), f32),
        pltpu.VMEM_SHARED((NPAD,), f32),
    ],
)


def _t_body(srcg_hbm, dstg_hbm, u_hbm, out_hbm,
            idx_s, idx_d, val_g, zer_v, t_sh):
    cid = lax.axis_index("c")
    sid = lax.axis_index("s")
    w = sid * NC + cid
    _fill(zer_v, RPT, 0.0)
    pltpu.sync_copy(zer_v, t_sh.at[pl.ds(sid * RPT, RPT)])
    plsc.subcore_barrier()

    def body(j, _):
        g = w * NGT + j
        pltpu.sync_copy(srcg_hbm.at[g], idx_s)
        pltpu.sync_copy(dstg_hbm.at[g], idx_d)
        pltpu.sync_copy(u_hbm.at[idx_s], val_g)
        pltpu.sync_copy(val_g, t_sh.at[idx_d], add=True)
        return 0
    lax.fori_loop(0, NGT, body, 0)
    plsc.subcore_barrier()
    pltpu.sync_copy(t_sh.at[pl.ds(sid * RPT, RPT)], zer_v)
    pltpu.sync_copy(zer_v, out_hbm.at[pl.ds(cid * NPAD + sid * RPT, RPT)])


_t_call = pl.kernel(
    _t_body,
    out_type=jax.ShapeDtypeStruct((NC * NPAD,), f32),
    mesh=_mesh,
    compiler_params=pltpu.CompilerParams(use_tc_tiling_on_sc=False),
    scratch_types=[
        pltpu.VMEM((KG * B,), i32),
        pltpu.VMEM((KG * B,), i32),
        pltpu.VMEM((KG * B,), f32),
        pltpu.VMEM((RPT,), f32),
        pltpu.VMEM_SHARED((NPAD,), f32),
    ],
)


def _row_body(srcc_hbm, dstc_hbm, ylo_hbm, yhi_hbm, out_hbm,
              isq, idq, st0, st1, zrow_v, a_sh, sg0, sg1, ss0, ss1):
    cid = lax.axis_index("c")
    sid = lax.axis_index("s")
    ZR = 32
    st = (st0, st1)
    sg = (sg0, sg1)
    ss = (ss0, ss1)

    def zinit(i, _):
        zrow_v[i, pl.ds(0, L)] = jnp.zeros((L,), f32)
        zrow_v[i, pl.ds(L, L)] = jnp.zeros((L,), f32)
        return 0
    lax.fori_loop(0, ZR, zinit, 0)
    for k in range(RPT // ZR):
        pltpu.sync_copy(zrow_v, a_sh.at[pl.ds(sid * RPT + k * ZR, ZR)])
    plsc.subcore_barrier()

    def gather(idxref, stref, sem):
        @pl.when(cid == 0)
        def _():
            pltpu.async_copy(ylo_hbm.at[idxref], stref, sem)

        @pl.when(cid == 1)
        def _():
            pltpu.async_copy(yhi_hbm.at[idxref], stref, sem)
        return pltpu.make_async_copy(ylo_hbm.at[idxref], stref, sem)

    def chunk(c, _):
        ci = sid * NCHR + c
        pltpu.sync_copy(srcc_hbm.at[ci], isq)
        pltpu.sync_copy(dstc_hbm.at[ci], idq)
        gd = [None, None]
        sd = [None, None]
        for k in range(CQR):
            slot = k % 2
            if k >= 2:
                sd[slot].wait()
            gd[slot] = gather(isq.at[k], st[slot], sg[slot])
            if k >= 1:
                gd[1 - slot].wait()
                sd[1 - slot] = pltpu.async_copy(
                    st[1 - slot], a_sh.at[idq.at[k - 1]], ss[1 - slot],
                    add=True)
        last = (CQR - 1) % 2
        gd[last].wait()
        sd[last] = pltpu.async_copy(
            st[last], a_sh.at[idq.at[CQR - 1]], ss[last], add=True)
        sd[0].wait()
        sd[1].wait()
        return 0
    lax.fori_loop(0, NCHR, chunk, 0)
    plsc.subcore_barrier()
    for k in range(RPT // ZR):
        pltpu.sync_copy(a_sh.at[pl.ds(sid * RPT + k * ZR, ZR)], zrow_v)
        pltpu.sync_copy(zrow_v,
                        out_hbm.at[pl.ds(cid * NPAD + sid * RPT + k * ZR, ZR)])


_row_call = pl.kernel(
    _row_body,
    out_type=jax.ShapeDtypeStruct((NC * NPAD, HID // 2), f32),
    mesh=_mesh,
    compiler_params=pltpu.CompilerParams(use_tc_tiling_on_sc=False),
    scratch_types=[
        pltpu.VMEM((CQR, GBR), i32),
        pltpu.VMEM((CQR, GBR), i32),
        pltpu.VMEM((GBR, HID // 2), f32),
        pltpu.VMEM((GBR, HID // 2), f32),
        pltpu.VMEM((32, HID // 2), f32),
        pltpu.VMEM_SHARED((NPAD, HID // 2), f32),
        pltpu.SemaphoreType.DMA,
        pltpu.SemaphoreType.DMA,
        pltpu.SemaphoreType.DMA,
        pltpu.SemaphoreType.DMA,
    ],
)


def _tc1_body(hist_ref, x_ref, d_ref, u_ref):
    deg = hist_ref[0] + hist_ref[1] + 1.0
    d = lax.rsqrt(deg)
    d_ref[...] = d
    u_ref[...] = d * x_ref[...]


_tc1 = pl.pallas_call(
    _tc1_body,
    grid=(NROW // 8,),
    in_specs=[
        pl.BlockSpec((NC, 8, NLANE), lambda i: (0, i, 0)),
        pl.BlockSpec((8, NLANE), lambda i: (i, 0)),
    ],
    out_specs=[
        pl.BlockSpec((8, NLANE), lambda i: (i, 0)),
        pl.BlockSpec((8, NLANE), lambda i: (i, 0)),
    ],
    out_shape=[
        jax.ShapeDtypeStruct((NROW, NLANE), f32),
        jax.ShapeDtypeStruct((NROW, NLANE), f32),
    ],
)


def _tc2_body(t_ref, d_ref, x_ref, s_ref):
    d = d_ref[...]
    s_ref[...] = d * (t_ref[0] + t_ref[1] + d * x_ref[...])


_tc2 = pl.pallas_call(
    _tc2_body,
    grid=(NROW // 8,),
    in_specs=[
        pl.BlockSpec((NC, 8, NLANE), lambda i: (0, i, 0)),
        pl.BlockSpec((8, NLANE), lambda i: (i, 0)),
        pl.BlockSpec((8, NLANE), lambda i: (i, 0)),
    ],
    out_specs=pl.BlockSpec((8, NLANE), lambda i: (i, 0)),
    out_shape=jax.ShapeDtypeStruct((NROW, NLANE), f32),
)

RB = 1024


def _tc3_body(s_ref, d_ref, w1_ref, b1_ref, pa_ref, w2_ref,
              z_ref, ylo_ref, yhi_ref):
    s = s_ref[...]
    h = s * w1_ref[...] + b1_ref[...]
    a = pa_ref[0, 0]
    h = jnp.where(h >= 0, h, a * h)
    z = jnp.dot(h, w2_ref[...], precision=lax.Precision.HIGHEST,
                preferred_element_type=f32)
    y = d_ref[...] * z
    z_ref[...] = z
    ylo_ref[...] = y[:, :HID // 2]
    yhi_ref[...] = y[:, HID // 2:]


_tc3 = pl.pallas_call(
    _tc3_body,
    grid=(NPAD // RB,),
    in_specs=[
        pl.BlockSpec((RB, 1), lambda i: (i, 0)),
        pl.BlockSpec((RB, 1), lambda i: (i, 0)),
        pl.BlockSpec((1, HID), lambda i: (0, 0)),
        pl.BlockSpec((1, HID), lambda i: (0, 0)),
        pl.BlockSpec(memory_space=pltpu.SMEM),
        pl.BlockSpec((HID, HID), lambda i: (0, 0)),
    ],
    out_specs=[
        pl.BlockSpec((RB, HID), lambda i: (i, 0)),
        pl.BlockSpec((RB, HID // 2), lambda i: (i, 0)),
        pl.BlockSpec((RB, HID // 2), lambda i: (i, 0)),
    ],
    out_shape=[
        jax.ShapeDtypeStruct((NPAD, HID), f32),
        jax.ShapeDtypeStruct((NPAD, HID // 2), f32),
        jax.ShapeDtypeStruct((NPAD, HID // 2), f32),
    ],
)


def _tc4_body(alo_ref, ahi_ref, z_ref, d_ref, b2_ref, out_ref):
    d = d_ref[...]
    z = z_ref[...]
    b2 = b2_ref[...]
    lo = d * (alo_ref[...] + d * z[:, :HID // 2]) + b2[:, :HID // 2]
    hi = d * (ahi_ref[...] + d * z[:, HID // 2:]) + b2[:, HID // 2:]
    out_ref[...] = jnp.concatenate([lo, hi], axis=1)


_tc4 = pl.pallas_call(
    _tc4_body,
    grid=(NPAD // RB,),
    in_specs=[
        pl.BlockSpec((RB, HID // 2), lambda i: (i, 0)),
        pl.BlockSpec((RB, HID // 2), lambda i: (i, 0)),
        pl.BlockSpec((RB, HID), lambda i: (i, 0)),
        pl.BlockSpec((RB, 1), lambda i: (i, 0)),
        pl.BlockSpec((1, HID), lambda i: (0, 0)),
    ],
    out_specs=pl.BlockSpec((RB, HID), lambda i: (i, 0)),
    out_shape=jax.ShapeDtypeStruct((NPAD, HID), f32),
)


def kernel(data_x, data_adj, W1, b1, prelu_a, W2, b2):
    x = data_x[:, 0].astype(f32)
    xp = jnp.pad(x, (0, NPAD - N))
    src = data_adj[0].astype(i32)
    dst = data_adj[1].astype(i32)
    src2 = jnp.pad(src, (0, E2 - E))
    dst2 = jnp.pad(dst, (0, E2 - E), constant_values=NPAD - 1)
    srcg = src2.reshape(E2 // (KG * B), KG * B)
    dstg = dst2.reshape(E2 // (KG * B), KG * B)
    srcc = src2.reshape(E2 // (CQR * GBR), CQR, GBR)
    dstc = dst2.reshape(E2 // (CQR * GBR), CQR, GBR)

    hist = _hist_call(dstg)
    hist2 = hist.reshape(NC, NROW, NLANE)
    x2 = xp.reshape(NROW, NLANE)
    d2, u2 = _tc1(hist2, x2)

    tpart = _t_call(srcg, dstg, u2.reshape(NPAD))
    s2 = _tc2(tpart.reshape(NC, NROW, NLANE), d2, x2)

    s_col = s2.reshape(NPAD, 1)
    d_col = d2.reshape(NPAD, 1)
    z, ylo, yhi = _tc3(s_col, d_col, W1.reshape(1, HID).astype(f32),
                       b1.reshape(1, HID).astype(f32),
                       prelu_a.reshape(1, 1).astype(f32), W2.astype(f32))

    apart = _row_call(srcc, dstc, ylo, yhi)
    outp = _tc4(apart[:NPAD], apart[NPAD:], z, d_col,
                b2.reshape(1, HID).astype(f32))
    return outp[:N]


# speedup vs baseline: 25.5787x; 1.0594x over previous
import functools

import jax
import jax.numpy as jnp
from jax import lax
from jax.experimental import pallas as pl
from jax.experimental.pallas import tpu as pltpu
from jax.experimental.pallas import tpu_sc as plsc

N = 50000
E = 800000
HID = 64
NPAD = 50176
NROW, NLANE = 392, 128
NC, NS, L = 2, 16, 16
NW = NC * NS
E2 = 819200
GBT = 800
CQT = 8
NCHT = E2 // (CQT * GBT * NW)
GBR = 320
CQR = 4
NCHR = E2 // (CQR * GBR * NS)
RPT = NPAD // NS

_mesh = plsc.VectorSubcoreMesh(core_axis_name="c", subcore_axis_name="s")
f32 = jnp.float32
i32 = jnp.int32


def _fill(ref, n, value):
    def body(i, _):
        ref[pl.ds(i * L, L)] = jnp.full((L,), value, f32)
        return 0
    lax.fori_loop(0, n // L, body, 0)


def _hist_body(dstt_hbm, out_hbm, idq0, idq1, ones_v, zer_v, hist_sh,
               sl0, sl1, ssc):
    cid = lax.axis_index("c")
    sid = lax.axis_index("s")
    w = sid * NC + cid
    idq = (idq0, idq1)
    sl = (sl0, sl1)

    _fill(ones_v, GBT, 1.0)
    _fill(zer_v, RPT, 0.0)
    pltpu.sync_copy(zer_v, hist_sh.at[pl.ds(sid * RPT, RPT)])
    plsc.subcore_barrier()

    pltpu.async_copy(dstt_hbm.at[w * NCHT], idq[0], sl[0])

    def do_chunk(c, slot):
        @pl.when(c + 1 < NCHT)
        def _():
            pltpu.async_copy(dstt_hbm.at[w * NCHT + c + 1],
                             idq[1 - slot], sl[1 - slot])
        pltpu.make_async_copy(dstt_hbm.at[w * NCHT], idq[slot],
                              sl[slot]).wait()
        sds = [pltpu.async_copy(ones_v, hist_sh.at[idq[slot].at[k]],
                                ssc, add=True)
               for k in range(CQT)]
        for dsc in sds:
            dsc.wait()

    def pair(i, _):
        do_chunk(2 * i, 0)
        do_chunk(2 * i + 1, 1)
        return 0
    lax.fori_loop(0, NCHT // 2, pair, 0)
    plsc.subcore_barrier()
    pltpu.sync_copy(hist_sh.at[pl.ds(sid * RPT, RPT)], zer_v)
    pltpu.sync_copy(zer_v, out_hbm.at[pl.ds(cid * NPAD + sid * RPT, RPT)])


_hist_call = pl.kernel(
    _hist_body,
    out_type=jax.ShapeDtypeStruct((NC * NPAD,), f32),
    mesh=_mesh,
    compiler_params=pltpu.CompilerParams(use_tc_tiling_on_sc=False),
    scratch_types=[
        pltpu.VMEM((CQT, GBT), i32),
        pltpu.VMEM((CQT, GBT), i32),
        pltpu.VMEM((GBT,), f32),
        pltpu.VMEM((RPT,), f32),
        pltpu.VMEM_SHARED((NPAD,), f32),
    ] + [pltpu.SemaphoreType.DMA] * 3,
)


def _t_body(srct_hbm, dstt_hbm, u_hbm, out_hbm,
            isq0, isq1, idq0, idq1, val0, val1, zer_v, t_sh,
            sa0, sa1, sb0, sb1, sg0, sg1, ss0, ss1):
    cid = lax.axis_index("c")
    sid = lax.axis_index("s")
    w = sid * NC + cid
    isq = (isq0, isq1)
    idq = (idq0, idq1)
    val = (val0, val1)
    sa = (sa0, sa1)
    sb = (sb0, sb1)
    sg = (sg0, sg1)
    ss = (ss0, ss1)
    _fill(zer_v, RPT, 0.0)
    pltpu.sync_copy(zer_v, t_sh.at[pl.ds(sid * RPT, RPT)])
    plsc.subcore_barrier()

    pltpu.async_copy(srct_hbm.at[w * NCHT], isq[0], sa[0])
    pltpu.async_copy(dstt_hbm.at[w * NCHT], idq[0], sb[0])

    def do_chunk(c, cs):
        @pl.when(c + 1 < NCHT)
        def _():
            pltpu.async_copy(srct_hbm.at[w * NCHT + c + 1],
                             isq[1 - cs], sa[1 - cs])
            pltpu.async_copy(dstt_hbm.at[w * NCHT + c + 1],
                             idq[1 - cs], sb[1 - cs])
        pltpu.make_async_copy(srct_hbm.at[w * NCHT], isq[cs], sa[cs]).wait()
        pltpu.make_async_copy(dstt_hbm.at[w * NCHT], idq[cs], sb[cs]).wait()
        gd = [None, None]
        sd = [None, None]
        for k in range(CQT):
            vs = k % 2
            if k >= 2:
                sd[vs].wait()
            gd[vs] = pltpu.async_copy(u_hbm.at[isq[cs].at[k]], val[vs],
                                      sg[vs])
            if k >= 1:
                gd[1 - vs].wait()
                sd[1 - vs] = pltpu.async_copy(
                    val[1 - vs], t_sh.at[idq[cs].at[k - 1]], ss[1 - vs],
                    add=True)
        lastv = (CQT - 1) % 2
        gd[lastv].wait()
        sd[lastv] = pltpu.async_copy(
            val[lastv], t_sh.at[idq[cs].at[CQT - 1]], ss[lastv], add=True)
        sd[0].wait()
        sd[1].wait()

    def pair(i, _):
        do_chunk(2 * i, 0)
        do_chunk(2 * i + 1, 1)
        return 0
    lax.fori_loop(0, NCHT // 2, pair, 0)
    plsc.subcore_barrier()
    pltpu.sync_copy(t_sh.at[pl.ds(sid * RPT, RPT)], zer_v)
    pltpu.sync_copy(zer_v, out_hbm.at[pl.ds(cid * NPAD + sid * RPT, RPT)])


_t_call = pl.kernel(
    _t_body,
    out_type=jax.ShapeDtypeStruct((NC * NPAD,), f32),
    mesh=_mesh,
    compiler_params=pltpu.CompilerParams(use_tc_tiling_on_sc=False),
    scratch_types=[
        pltpu.VMEM((CQT, GBT), i32),
        pltpu.VMEM((CQT, GBT), i32),
        pltpu.VMEM((CQT, GBT), i32),
        pltpu.VMEM((CQT, GBT), i32),
        pltpu.VMEM((GBT,), f32),
        pltpu.VMEM((GBT,), f32),
        pltpu.VMEM((RPT,), f32),
        pltpu.VMEM_SHARED((NPAD,), f32),
    ] + [pltpu.SemaphoreType.DMA] * 8,
)


def _row_body(srcc_hbm, dstc_hbm, ylo_hbm, yhi_hbm, out_hbm,
              isq0, isq1, idq0, idq1, st0, st1, zrow_v, a_sh,
              sa0, sa1, sb0, sb1, sg0, sg1, ss0, ss1):
    cid = lax.axis_index("c")
    sid = lax.axis_index("s")
    ZR = 32
    isq = (isq0, isq1)
    idq = (idq0, idq1)
    st = (st0, st1)
    sa = (sa0, sa1)
    sb = (sb0, sb1)
    sg = (sg0, sg1)
    ss = (ss0, ss1)

    def zinit(i, _):
        zrow_v[i, pl.ds(0, L)] = jnp.zeros((L,), f32)
        zrow_v[i, pl.ds(L, L)] = jnp.zeros((L,), f32)
        return 0
    lax.fori_loop(0, ZR, zinit, 0)
    for k in range(RPT // ZR):
        pltpu.sync_copy(zrow_v, a_sh.at[pl.ds(sid * RPT + k * ZR, ZR)])
    plsc.subcore_barrier()

    def gather(idxref, stref, sem):
        @pl.when(cid == 0)
        def _():
            pltpu.async_copy(ylo_hbm.at[idxref], stref, sem)

        @pl.when(cid == 1)
        def _():
            pltpu.async_copy(yhi_hbm.at[idxref], stref, sem)
        return pltpu.make_async_copy(ylo_hbm.at[idxref], stref, sem)

    pltpu.async_copy(srcc_hbm.at[sid * NCHR], isq[0], sa[0])
    pltpu.async_copy(dstc_hbm.at[sid * NCHR], idq[0], sb[0])

    def do_chunk(c, cs):
        @pl.when(c + 1 < NCHR)
        def _():
            pltpu.async_copy(srcc_hbm.at[sid * NCHR + c + 1],
                             isq[1 - cs], sa[1 - cs])
            pltpu.async_copy(dstc_hbm.at[sid * NCHR + c + 1],
                             idq[1 - cs], sb[1 - cs])
        pltpu.make_async_copy(srcc_hbm.at[sid * NCHR], isq[cs], sa[cs]).wait()
        pltpu.make_async_copy(dstc_hbm.at[sid * NCHR], idq[cs], sb[cs]).wait()
        gd = [None, None]
        sd = [None, None]
        for k in range(CQR):
            slot = k % 2
            if k >= 2:
                sd[slot].wait()
            gd[slot] = gather(isq[cs].at[k], st[slot], sg[slot])
            if k >= 1:
                gd[1 - slot].wait()
                sd[1 - slot] = pltpu.async_copy(
                    st[1 - slot], a_sh.at[idq[cs].at[k - 1]], ss[1 - slot],
                    add=True)
        last = (CQR - 1) % 2
        gd[last].wait()
        sd[last] = pltpu.async_copy(
            st[last], a_sh.at[idq[cs].at[CQR - 1]], ss[last], add=True)
        sd[0].wait()
        sd[1].wait()

    def pair(i, _):
        do_chunk(2 * i, 0)
        do_chunk(2 * i + 1, 1)
        return 0
    lax.fori_loop(0, NCHR // 2, pair, 0)
    plsc.subcore_barrier()
    for k in range(RPT // ZR):
        pltpu.sync_copy(a_sh.at[pl.ds(sid * RPT + k * ZR, ZR)], zrow_v)
        pltpu.sync_copy(zrow_v,
                        out_hbm.at[pl.ds(cid * NPAD + sid * RPT + k * ZR, ZR)])


_row_call = pl.kernel(
    _row_body,
    out_type=jax.ShapeDtypeStruct((NC * NPAD, HID // 2), f32),
    mesh=_mesh,
    compiler_params=pltpu.CompilerParams(use_tc_tiling_on_sc=False),
    scratch_types=[
        pltpu.VMEM((CQR, GBR), i32),
        pltpu.VMEM((CQR, GBR), i32),
        pltpu.VMEM((CQR, GBR), i32),
        pltpu.VMEM((CQR, GBR), i32),
        pltpu.VMEM((GBR, HID // 2), f32),
        pltpu.VMEM((GBR, HID // 2), f32),
        pltpu.VMEM((32, HID // 2), f32),
        pltpu.VMEM_SHARED((NPAD, HID // 2), f32),
    ] + [pltpu.SemaphoreType.DMA] * 8,
)


def _tc1_body(hist_ref, x_ref, d_ref, u_ref):
    deg = hist_ref[0] + hist_ref[1] + 1.0
    d = lax.rsqrt(deg)
    d_ref[...] = d
    u_ref[...] = d * x_ref[...]


_tc1 = pl.pallas_call(
    _tc1_body,
    grid=(NROW // 8,),
    in_specs=[
        pl.BlockSpec((NC, 8, NLANE), lambda i: (0, i, 0)),
        pl.BlockSpec((8, NLANE), lambda i: (i, 0)),
    ],
    out_specs=[
        pl.BlockSpec((8, NLANE), lambda i: (i, 0)),
        pl.BlockSpec((8, NLANE), lambda i: (i, 0)),
    ],
    out_shape=[
        jax.ShapeDtypeStruct((NROW, NLANE), f32),
        jax.ShapeDtypeStruct((NROW, NLANE), f32),
    ],
)


def _tc2_body(t_ref, d_ref, x_ref, s_ref):
    d = d_ref[...]
    s_ref[...] = d * (t_ref[0] + t_ref[1] + d * x_ref[...])


_tc2 = pl.pallas_call(
    _tc2_body,
    grid=(NROW // 8,),
    in_specs=[
        pl.BlockSpec((NC, 8, NLANE), lambda i: (0, i, 0)),
        pl.BlockSpec((8, NLANE), lambda i: (i, 0)),
        pl.BlockSpec((8, NLANE), lambda i: (i, 0)),
    ],
    out_specs=pl.BlockSpec((8, NLANE), lambda i: (i, 0)),
    out_shape=jax.ShapeDtypeStruct((NROW, NLANE), f32),
)

RB = 1024


def _tc3_body(s_ref, d_ref, w1_ref, b1_ref, pa_ref, w2_ref,
              z_ref, ylo_ref, yhi_ref):
    s = s_ref[...]
    h = s * w1_ref[...] + b1_ref[...]
    a = pa_ref[0, 0]
    h = jnp.where(h >= 0, h, a * h)
    z = jnp.dot(h, w2_ref[...], precision=lax.Precision.HIGHEST,
                preferred_element_type=f32)
    y = d_ref[...] * z
    z_ref[...] = z
    ylo_ref[...] = y[:, :HID // 2]
    yhi_ref[...] = y[:, HID // 2:]


_tc3 = pl.pallas_call(
    _tc3_body,
    grid=(NPAD // RB,),
    in_specs=[
        pl.BlockSpec((RB, 1), lambda i: (i, 0)),
        pl.BlockSpec((RB, 1), lambda i: (i, 0)),
        pl.BlockSpec((1, HID), lambda i: (0, 0)),
        pl.BlockSpec((1, HID), lambda i: (0, 0)),
        pl.BlockSpec(memory_space=pltpu.SMEM),
        pl.BlockSpec((HID, HID), lambda i: (0, 0)),
    ],
    out_specs=[
        pl.BlockSpec((RB, HID), lambda i: (i, 0)),
        pl.BlockSpec((RB, HID // 2), lambda i: (i, 0)),
        pl.BlockSpec((RB, HID // 2), lambda i: (i, 0)),
    ],
    out_shape=[
        jax.ShapeDtypeStruct((NPAD, HID), f32),
        jax.ShapeDtypeStruct((NPAD, HID // 2), f32),
        jax.ShapeDtypeStruct((NPAD, HID // 2), f32),
    ],
)


def _tc4_body(alo_ref, ahi_ref, z_ref, d_ref, b2_ref, out_ref):
    d = d_ref[...]
    z = z_ref[...]
    b2 = b2_ref[...]
    lo = d * (alo_ref[...] + d * z[:, :HID // 2]) + b2[:, :HID // 2]
    hi = d * (ahi_ref[...] + d * z[:, HID // 2:]) + b2[:, HID // 2:]
    out_ref[...] = jnp.concatenate([lo, hi], axis=1)


_tc4 = pl.pallas_call(
    _tc4_body,
    grid=(NPAD // RB,),
    in_specs=[
        pl.BlockSpec((RB, HID // 2), lambda i: (i, 0)),
        pl.BlockSpec((RB, HID // 2), lambda i: (i, 0)),
        pl.BlockSpec((RB, HID), lambda i: (i, 0)),
        pl.BlockSpec((RB, 1), lambda i: (i, 0)),
        pl.BlockSpec((1, HID), lambda i: (0, 0)),
    ],
    out_specs=pl.BlockSpec((RB, HID), lambda i: (i, 0)),
    out_shape=jax.ShapeDtypeStruct((NPAD, HID), f32),
)


def kernel(data_x, data_adj, W1, b1, prelu_a, W2, b2):
    x = data_x[:, 0].astype(f32)
    xp = jnp.pad(x, (0, NPAD - N))
    src = data_adj[0].astype(i32)
    dst = data_adj[1].astype(i32)
    src2 = jnp.pad(src, (0, E2 - E))
    dst2 = jnp.pad(dst, (0, E2 - E), constant_values=NPAD - 1)
    srct = src2.reshape(E2 // (CQT * GBT), CQT, GBT)
    dstt = dst2.reshape(E2 // (CQT * GBT), CQT, GBT)
    srcc = src2.reshape(E2 // (CQR * GBR), CQR, GBR)
    dstc = dst2.reshape(E2 // (CQR * GBR), CQR, GBR)

    hist = _hist_call(dstt)
    hist2 = hist.reshape(NC, NROW, NLANE)
    x2 = xp.reshape(NROW, NLANE)
    d2, u2 = _tc1(hist2, x2)

    tpart = _t_call(srct, dstt, u2.reshape(NPAD))
    s2 = _tc2(tpart.reshape(NC, NROW, NLANE), d2, x2)

    s_col = s2.reshape(NPAD, 1)
    d_col = d2.reshape(NPAD, 1)
    z, ylo, yhi = _tc3(s_col, d_col, W1.reshape(1, HID).astype(f32),
                       b1.reshape(1, HID).astype(f32),
                       prelu_a.reshape(1, 1).astype(f32), W2.astype(f32))

    apart = _row_call(srcc, dstc, ylo, yhi)
    outp = _tc4(apart[:NPAD], apart[NPAD:], z, d_col,
                b2.reshape(1, HID).astype(f32))
    return outp[:N]


# speedup vs baseline: 32.4121x; 1.2672x over previous
import functools

import jax
import jax.numpy as jnp
from jax import lax
from jax.experimental import pallas as pl
from jax.experimental.pallas import tpu as pltpu
from jax.experimental.pallas import tpu_sc as plsc

N = 50000
E = 800000
HID = 64
NPAD = 50176
NROW, NLANE = 392, 128
NC, NS, L = 2, 16, 16
NW = NC * NS
E2 = 819200
GBT = 800
CQT = 8
NCHT = E2 // (CQT * GBT * NW)
GBR = 1024
CQR = 5
NCHR = E2 // (CQR * GBR * NS)
RPT = NPAD // NS

bf16 = jnp.bfloat16
_mesh = plsc.VectorSubcoreMesh(core_axis_name="c", subcore_axis_name="s")
f32 = jnp.float32
i32 = jnp.int32


def _fill(ref, n, value):
    def body(i, _):
        ref[pl.ds(i * L, L)] = jnp.full((L,), value, f32)
        return 0
    lax.fori_loop(0, n // L, body, 0)


def _hist_body(dstt_hbm, out_hbm, idq0, idq1, ones_v, zer_v, hist_sh,
               sl0, sl1, ssc):
    cid = lax.axis_index("c")
    sid = lax.axis_index("s")
    w = sid * NC + cid
    idq = (idq0, idq1)
    sl = (sl0, sl1)

    _fill(ones_v, GBT, 1.0)
    _fill(zer_v, RPT, 0.0)
    pltpu.sync_copy(zer_v, hist_sh.at[pl.ds(sid * RPT, RPT)])
    plsc.subcore_barrier()

    pltpu.async_copy(dstt_hbm.at[w * NCHT], idq[0], sl[0])

    def do_chunk(c, slot):
        @pl.when(c + 1 < NCHT)
        def _():
            pltpu.async_copy(dstt_hbm.at[w * NCHT + c + 1],
                             idq[1 - slot], sl[1 - slot])
        pltpu.make_async_copy(dstt_hbm.at[w * NCHT], idq[slot],
                              sl[slot]).wait()
        sds = [pltpu.async_copy(ones_v, hist_sh.at[idq[slot].at[k]],
                                ssc, add=True)
               for k in range(CQT)]
        for dsc in sds:
            dsc.wait()

    def pair(i, _):
        do_chunk(2 * i, 0)
        do_chunk(2 * i + 1, 1)
        return 0
    lax.fori_loop(0, NCHT // 2, pair, 0)
    plsc.subcore_barrier()
    pltpu.sync_copy(hist_sh.at[pl.ds(sid * RPT, RPT)], zer_v)
    pltpu.sync_copy(zer_v, out_hbm.at[pl.ds(cid * NPAD + sid * RPT, RPT)])


_hist_call = pl.kernel(
    _hist_body,
    out_type=jax.ShapeDtypeStruct((NC * NPAD,), f32),
    mesh=_mesh,
    compiler_params=pltpu.CompilerParams(use_tc_tiling_on_sc=False),
    scratch_types=[
        pltpu.VMEM((CQT, GBT), i32),
        pltpu.VMEM((CQT, GBT), i32),
        pltpu.VMEM((GBT,), f32),
        pltpu.VMEM((RPT,), f32),
        pltpu.VMEM_SHARED((NPAD,), f32),
    ] + [pltpu.SemaphoreType.DMA] * 3,
)


def _t_body(srct_hbm, dstt_hbm, u_hbm, out_hbm,
            isq0, isq1, idq0, idq1, val0, val1, zer_v, t_sh,
            sa0, sa1, sb0, sb1, sg0, sg1, ss0, ss1):
    cid = lax.axis_index("c")
    sid = lax.axis_index("s")
    w = sid * NC + cid
    isq = (isq0, isq1)
    idq = (idq0, idq1)
    val = (val0, val1)
    sa = (sa0, sa1)
    sb = (sb0, sb1)
    sg = (sg0, sg1)
    ss = (ss0, ss1)
    _fill(zer_v, RPT, 0.0)
    pltpu.sync_copy(zer_v, t_sh.at[pl.ds(sid * RPT, RPT)])
    plsc.subcore_barrier()

    pltpu.async_copy(srct_hbm.at[w * NCHT], isq[0], sa[0])
    pltpu.async_copy(dstt_hbm.at[w * NCHT], idq[0], sb[0])

    def do_chunk(c, cs):
        @pl.when(c + 1 < NCHT)
        def _():
            pltpu.async_copy(srct_hbm.at[w * NCHT + c + 1],
                             isq[1 - cs], sa[1 - cs])
            pltpu.async_copy(dstt_hbm.at[w * NCHT + c + 1],
                             idq[1 - cs], sb[1 - cs])
        pltpu.make_async_copy(srct_hbm.at[w * NCHT], isq[cs], sa[cs]).wait()
        pltpu.make_async_copy(dstt_hbm.at[w * NCHT], idq[cs], sb[cs]).wait()
        gd = [None, None]
        sd = [None, None]
        for k in range(CQT):
            vs = k % 2
            if k >= 2:
                sd[vs].wait()
            gd[vs] = pltpu.async_copy(u_hbm.at[isq[cs].at[k]], val[vs],
                                      sg[vs])
            if k >= 1:
                gd[1 - vs].wait()
                sd[1 - vs] = pltpu.async_copy(
                    val[1 - vs], t_sh.at[idq[cs].at[k - 1]], ss[1 - vs],
                    add=True)
        lastv = (CQT - 1) % 2
        gd[lastv].wait()
        sd[lastv] = pltpu.async_copy(
            val[lastv], t_sh.at[idq[cs].at[CQT - 1]], ss[lastv], add=True)
        sd[0].wait()
        sd[1].wait()

    def pair(i, _):
        do_chunk(2 * i, 0)
        do_chunk(2 * i + 1, 1)
        return 0
    lax.fori_loop(0, NCHT // 2, pair, 0)
    plsc.subcore_barrier()
    pltpu.sync_copy(t_sh.at[pl.ds(sid * RPT, RPT)], zer_v)
    pltpu.sync_copy(zer_v, out_hbm.at[pl.ds(cid * NPAD + sid * RPT, RPT)])


_t_call = pl.kernel(
    _t_body,
    out_type=jax.ShapeDtypeStruct((NC * NPAD,), f32),
    mesh=_mesh,
    compiler_params=pltpu.CompilerParams(use_tc_tiling_on_sc=False),
    scratch_types=[
        pltpu.VMEM((CQT, GBT), i32),
        pltpu.VMEM((CQT, GBT), i32),
        pltpu.VMEM((CQT, GBT), i32),
        pltpu.VMEM((CQT, GBT), i32),
        pltpu.VMEM((GBT,), f32),
        pltpu.VMEM((GBT,), f32),
        pltpu.VMEM((RPT,), f32),
        pltpu.VMEM_SHARED((NPAD,), f32),
    ] + [pltpu.SemaphoreType.DMA] * 8,
)


def _row_body(srcc_hbm, dstc_hbm, ylo_hbm, yhi_hbm, out_hbm,
              isq0, isq1, idq0, idq1, st0, st1, zrow_v, a_sh,
              sa0, sa1, sb0, sb1, sg0, sg1, ss0, ss1):
    cid = lax.axis_index("c")
    sid = lax.axis_index("s")
    ZR = 98
    isq = (isq0, isq1)
    idq = (idq0, idq1)
    st = (st0, st1)
    sa = (sa0, sa1)
    sb = (sb0, sb1)
    sg = (sg0, sg1)
    ss = (ss0, ss1)

    def zinit(i, _):
        zrow_v[i, pl.ds(0, 2 * L)] = jnp.zeros((2 * L,), bf16)
        return 0
    lax.fori_loop(0, ZR, zinit, 0)
    for k in range(RPT // ZR):
        pltpu.sync_copy(zrow_v, a_sh.at[pl.ds(sid * RPT + k * ZR, ZR)])
    plsc.subcore_barrier()

    def gather(idxref, stref, sem):
        @pl.when(cid == 0)
        def _():
            pltpu.async_copy(ylo_hbm.at[idxref], stref, sem)

        @pl.when(cid == 1)
        def _():
            pltpu.async_copy(yhi_hbm.at[idxref], stref, sem)
        return pltpu.make_async_copy(ylo_hbm.at[idxref], stref, sem)

    pltpu.async_copy(srcc_hbm.at[sid * NCHR], isq[0], sa[0])
    pltpu.async_copy(dstc_hbm.at[sid * NCHR], idq[0], sb[0])

    def do_chunk(c, cs):
        @pl.when(c + 1 < NCHR)
        def _():
            pltpu.async_copy(srcc_hbm.at[sid * NCHR + c + 1],
                             isq[1 - cs], sa[1 - cs])
            pltpu.async_copy(dstc_hbm.at[sid * NCHR + c + 1],
                             idq[1 - cs], sb[1 - cs])
        pltpu.make_async_copy(srcc_hbm.at[sid * NCHR], isq[cs], sa[cs]).wait()
        pltpu.make_async_copy(dstc_hbm.at[sid * NCHR], idq[cs], sb[cs]).wait()
        gd = [None, None]
        sd = [None, None]
        for k in range(CQR):
            slot = k % 2
            if k >= 2:
                sd[slot].wait()
            gd[slot] = gather(isq[cs].at[k], st[slot], sg[slot])
            if k >= 1:
                gd[1 - slot].wait()
                sd[1 - slot] = pltpu.async_copy(
                    st[1 - slot], a_sh.at[idq[cs].at[k - 1]], ss[1 - slot],
                    add=True)
        last = (CQR - 1) % 2
        gd[last].wait()
        sd[last] = pltpu.async_copy(
            st[last], a_sh.at[idq[cs].at[CQR - 1]], ss[last], add=True)
        sd[0].wait()
        sd[1].wait()

    def pair(i, _):
        do_chunk(2 * i, 0)
        do_chunk(2 * i + 1, 1)
        return 0
    lax.fori_loop(0, NCHR // 2, pair, 0)
    plsc.subcore_barrier()
    for k in range(RPT // ZR):
        pltpu.sync_copy(a_sh.at[pl.ds(sid * RPT + k * ZR, ZR)], zrow_v)
        pltpu.sync_copy(zrow_v,
                        out_hbm.at[pl.ds(cid * NPAD + sid * RPT + k * ZR, ZR)])


_row_call = pl.kernel(
    _row_body,
    out_type=jax.ShapeDtypeStruct((NC * NPAD, HID // 2), bf16),
    mesh=_mesh,
    compiler_params=pltpu.CompilerParams(use_tc_tiling_on_sc=False),
    scratch_types=[
        pltpu.VMEM((CQR, GBR), i32),
        pltpu.VMEM((CQR, GBR), i32),
        pltpu.VMEM((CQR, GBR), i32),
        pltpu.VMEM((CQR, GBR), i32),
        pltpu.VMEM((GBR, HID // 2), bf16),
        pltpu.VMEM((GBR, HID // 2), bf16),
        pltpu.VMEM((98, HID // 2), bf16),
        pltpu.VMEM_SHARED((NPAD, HID // 2), bf16),
    ] + [pltpu.SemaphoreType.DMA] * 8,
)


def _tc1_body(hist_ref, x_ref, d_ref, u_ref):
    deg = hist_ref[0] + hist_ref[1] + 1.0
    d = lax.rsqrt(deg)
    d_ref[...] = d
    u_ref[...] = d * x_ref[...]


_tc1 = pl.pallas_call(
    _tc1_body,
    grid=(NROW // 8,),
    in_specs=[
        pl.BlockSpec((NC, 8, NLANE), lambda i: (0, i, 0)),
        pl.BlockSpec((8, NLANE), lambda i: (i, 0)),
    ],
    out_specs=[
        pl.BlockSpec((8, NLANE), lambda i: (i, 0)),
        pl.BlockSpec((8, NLANE), lambda i: (i, 0)),
    ],
    out_shape=[
        jax.ShapeDtypeStruct((NROW, NLANE), f32),
        jax.ShapeDtypeStruct((NROW, NLANE), f32),
    ],
)


def _tc2_body(t_ref, d_ref, x_ref, s_ref):
    d = d_ref[...]
    s_ref[...] = d * (t_ref[0] + t_ref[1] + d * x_ref[...])


_tc2 = pl.pallas_call(
    _tc2_body,
    grid=(NROW // 8,),
    in_specs=[
        pl.BlockSpec((NC, 8, NLANE), lambda i: (0, i, 0)),
        pl.BlockSpec((8, NLANE), lambda i: (i, 0)),
        pl.BlockSpec((8, NLANE), lambda i: (i, 0)),
    ],
    out_specs=pl.BlockSpec((8, NLANE), lambda i: (i, 0)),
    out_shape=jax.ShapeDtypeStruct((NROW, NLANE), f32),
)

RB = 1024


def _tc3_body(t_ref, d_ref, x_ref, w1_ref, b1_ref, pa_ref, w2_ref,
              z_ref, ylo_ref, yhi_ref):
    d = d_ref[...]
    s = d * (t_ref[0] + t_ref[1] + d * x_ref[...])
    h = s * w1_ref[...] + b1_ref[...]
    a = pa_ref[0, 0]
    h = jnp.where(h >= 0, h, a * h)
    z = jnp.dot(h, w2_ref[...], precision=lax.Precision.HIGHEST,
                preferred_element_type=f32)
    y = (d * z).astype(bf16)
    z_ref[...] = z
    ylo_ref[...] = y[:, :HID // 2]
    yhi_ref[...] = y[:, HID // 2:]


_tc3 = pl.pallas_call(
    _tc3_body,
    grid=(NPAD // RB,),
    in_specs=[
        pl.BlockSpec((NC, RB, 1), lambda i: (0, i, 0)),
        pl.BlockSpec((RB, 1), lambda i: (i, 0)),
        pl.BlockSpec((RB, 1), lambda i: (i, 0)),
        pl.BlockSpec((1, HID), lambda i: (0, 0)),
        pl.BlockSpec((1, HID), lambda i: (0, 0)),
        pl.BlockSpec(memory_space=pltpu.SMEM),
        pl.BlockSpec((HID, HID), lambda i: (0, 0)),
    ],
    out_specs=[
        pl.BlockSpec((RB, HID), lambda i: (i, 0)),
        pl.BlockSpec((RB, HID // 2), lambda i: (i, 0)),
        pl.BlockSpec((RB, HID // 2), lambda i: (i, 0)),
    ],
    out_shape=[
        jax.ShapeDtypeStruct((NPAD, HID), f32),
        jax.ShapeDtypeStruct((NPAD, HID // 2), bf16),
        jax.ShapeDtypeStruct((NPAD, HID // 2), bf16),
    ],
)


def _tc4_body(alo_ref, ahi_ref, z_ref, d_ref, b2_ref, out_ref):
    d = d_ref[...]
    z = z_ref[...]
    b2 = b2_ref[...]
    alo = alo_ref[...].astype(f32)
    ahi = ahi_ref[...].astype(f32)
    lo = d * (alo + d * z[:, :HID // 2]) + b2[:, :HID // 2]
    hi = d * (ahi + d * z[:, HID // 2:]) + b2[:, HID // 2:]
    out_ref[...] = jnp.concatenate([lo, hi], axis=1)


_tc4 = pl.pallas_call(
    _tc4_body,
    grid=(NPAD // RB,),
    in_specs=[
        pl.BlockSpec((RB, HID // 2), lambda i: (i, 0)),
        pl.BlockSpec((RB, HID // 2), lambda i: (i, 0)),
        pl.BlockSpec((RB, HID), lambda i: (i, 0)),
        pl.BlockSpec((RB, 1), lambda i: (i, 0)),
        pl.BlockSpec((1, HID), lambda i: (0, 0)),
    ],
    out_specs=pl.BlockSpec((RB, HID), lambda i: (i, 0)),
    out_shape=jax.ShapeDtypeStruct((NPAD, HID), f32),
)


def kernel(data_x, data_adj, W1, b1, prelu_a, W2, b2):
    x = data_x[:, 0].astype(f32)
    xp = jnp.pad(x, (0, NPAD - N))
    src = data_adj[0].astype(i32)
    dst = data_adj[1].astype(i32)
    src2 = jnp.pad(src, (0, E2 - E))
    dst2 = jnp.pad(dst, (0, E2 - E), constant_values=NPAD - 1)
    srct = src2.reshape(E2 // (CQT * GBT), CQT, GBT)
    dstt = dst2.reshape(E2 // (CQT * GBT), CQT, GBT)
    srcc = src2.reshape(E2 // (CQR * GBR), CQR, GBR)
    dstc = dst2.reshape(E2 // (CQR * GBR), CQR, GBR)

    hist = _hist_call(dstt)
    hist2 = hist.reshape(NC, NROW, NLANE)
    x2 = xp.reshape(NROW, NLANE)
    d2, u2 = _tc1(hist2, x2)

    tpart = _t_call(srct, dstt, u2.reshape(NPAD))

    t_col = tpart.reshape(NC, NPAD, 1)
    x_col = xp.reshape(NPAD, 1)
    d_col = d2.reshape(NPAD, 1)
    z, ylo, yhi = _tc3(t_col, d_col, x_col, W1.reshape(1, HID).astype(f32),
                       b1.reshape(1, HID).astype(f32),
                       prelu_a.reshape(1, 1).astype(f32), W2.astype(f32))

    apart = _row_call(srcc, dstc, ylo, yhi)
    outp = _tc4(apart[:NPAD], apart[NPAD:], z, d_col,
                b2.reshape(1, HID).astype(f32))
    return outp[:N]


# speedup vs baseline: 33.3096x; 1.0277x over previous
import functools

import jax
import jax.numpy as jnp
from jax import lax
from jax.experimental import pallas as pl
from jax.experimental.pallas import tpu as pltpu
from jax.experimental.pallas import tpu_sc as plsc

N = 50000
E = 800000
HID = 64
NPAD = 50176
NROW, NLANE = 392, 128
NC, NS, L = 2, 16, 16
NW = NC * NS
E2 = 819200
GBT = 800
CQT = 8
NCHT = E2 // (CQT * GBT * NW)
GBR = 1024
CQR = 5
NCHR = E2 // (CQR * GBR * NS)
RPT = NPAD // NS

bf16 = jnp.bfloat16
_mesh = plsc.VectorSubcoreMesh(core_axis_name="c", subcore_axis_name="s")
f32 = jnp.float32
i32 = jnp.int32


def _fill(ref, n, value):
    def body(i, _):
        ref[pl.ds(i * L, L)] = jnp.full((L,), value, f32)
        return 0
    lax.fori_loop(0, n // L, body, 0)


NCHA = E2 // (CQT * GBT * NS)


def _scal_body(srct_hbm, dstt_hbm, x_hbm, d_hbm, u_hbm, t_hbm,
               isq0, isq1, idq0, idq1, val0, val1, zer_v, buf_v, acc_sh,
               sa0, sa1, sb0, sb1, sg0, sg1, ss0, ss1):
    cid = lax.axis_index("c")
    sid = lax.axis_index("s")
    w = sid * NC + cid
    isq = (isq0, isq1)
    idq = (idq0, idq1)
    val = (val0, val1)
    sa = (sa0, sa1)
    sb = (sb0, sb1)
    sg = (sg0, sg1)
    ss = (ss0, ss1)
    ones_v = val0

    _fill(ones_v, GBT, 1.0)
    _fill(zer_v, RPT, 0.0)
    pltpu.sync_copy(zer_v, acc_sh.at[pl.ds(sid * RPT, RPT)])
    plsc.subcore_barrier()

    pltpu.async_copy(dstt_hbm.at[sid * NCHA], idq[0], sb[0])

    def hist_chunk(c, slot):
        @pl.when(c + 1 < NCHA)
        def _():
            pltpu.async_copy(dstt_hbm.at[sid * NCHA + c + 1],
                             idq[1 - slot], sb[1 - slot])
        pltpu.make_async_copy(dstt_hbm.at[sid * NCHA], idq[slot],
                              sb[slot]).wait()
        sds = [pltpu.async_copy(ones_v, acc_sh.at[idq[slot].at[k]],
                                ss[0], add=True)
               for k in range(CQT)]
        for dsc in sds:
            dsc.wait()

    def hist_pair(i, _):
        hist_chunk(2 * i, 0)
        hist_chunk(2 * i + 1, 1)
        return 0
    lax.fori_loop(0, NCHA // 2, hist_pair, 0)
    plsc.subcore_barrier()

    pltpu.sync_copy(acc_sh.at[pl.ds(sid * RPT, RPT)], zer_v)
    pltpu.sync_copy(x_hbm.at[pl.ds(sid * RPT, RPT)], buf_v)

    def rsqrt_vec(i, _):
        deg = zer_v[pl.ds(i * L, L)] + 1.0
        bits = plsc.bitcast(deg, i32)
        y = plsc.bitcast(jnp.full((L,), 0x5f3759df, i32)
                         - lax.shift_right_logical(bits, 1), f32)
        half = 0.5 * deg
        y = y * (1.5 - half * y * y)
        y = y * (1.5 - half * y * y)
        y = y * (1.5 - half * y * y)
        y = y * (1.5 - half * y * y)
        x16 = buf_v[pl.ds(i * L, L)]
        zer_v[pl.ds(i * L, L)] = y
        buf_v[pl.ds(i * L, L)] = y * x16
        return 0
    lax.fori_loop(0, RPT // L, rsqrt_vec, 0)

    @pl.when(cid == 0)
    def _():
        pltpu.sync_copy(zer_v, d_hbm.at[pl.ds(sid * RPT, RPT)])
    pltpu.sync_copy(buf_v, u_hbm.at[pl.ds(cid * NPAD + sid * RPT, RPT)])
    _fill(zer_v, RPT, 0.0)
    pltpu.sync_copy(zer_v, acc_sh.at[pl.ds(sid * RPT, RPT)])
    plsc.subcore_barrier()

    utab = u_hbm.at[pl.ds(cid * NPAD, NPAD)]
    pltpu.async_copy(srct_hbm.at[w * NCHT], isq[0], sa[0])
    pltpu.async_copy(dstt_hbm.at[w * NCHT], idq[0], sb[0])

    def t_chunk(c, cs):
        @pl.when(c + 1 < NCHT)
        def _():
            pltpu.async_copy(srct_hbm.at[w * NCHT + c + 1],
                             isq[1 - cs], sa[1 - cs])
            pltpu.async_copy(dstt_hbm.at[w * NCHT + c + 1],
                             idq[1 - cs], sb[1 - cs])
        pltpu.make_async_copy(srct_hbm.at[w * NCHT], isq[cs], sa[cs]).wait()
        pltpu.make_async_copy(dstt_hbm.at[w * NCHT], idq[cs], sb[cs]).wait()
        gd = [None, None]
        sd = [None, None]
        for k in range(CQT):
            vs = k % 2
            if k >= 2:
                sd[vs].wait()
            gd[vs] = pltpu.async_copy(utab.at[isq[cs].at[k]], val[vs],
                                      sg[vs])
            if k >= 1:
                gd[1 - vs].wait()
                sd[1 - vs] = pltpu.async_copy(
                    val[1 - vs], acc_sh.at[idq[cs].at[k - 1]], ss[1 - vs],
                    add=True)
        lastv = (CQT - 1) % 2
        gd[lastv].wait()
        sd[lastv] = pltpu.async_copy(
            val[lastv], acc_sh.at[idq[cs].at[CQT - 1]], ss[lastv], add=True)
        sd[0].wait()
        sd[1].wait()

    def t_pair(i, _):
        t_chunk(2 * i, 0)
        t_chunk(2 * i + 1, 1)
        return 0
    lax.fori_loop(0, NCHT // 2, t_pair, 0)
    plsc.subcore_barrier()
    pltpu.sync_copy(acc_sh.at[pl.ds(sid * RPT, RPT)], zer_v)
    pltpu.sync_copy(zer_v, t_hbm.at[pl.ds(cid * NPAD + sid * RPT, RPT)])


_scal_call = pl.kernel(
    _scal_body,
    out_type=[
        jax.ShapeDtypeStruct((NPAD,), f32),
        jax.ShapeDtypeStruct((NC * NPAD,), f32),
        jax.ShapeDtypeStruct((NC * NPAD,), f32),
    ],
    mesh=_mesh,
    compiler_params=pltpu.CompilerParams(use_tc_tiling_on_sc=False,
                                         needs_layout_passes=False),
    scratch_types=[
        pltpu.VMEM((CQT, GBT), i32),
        pltpu.VMEM((CQT, GBT), i32),
        pltpu.VMEM((CQT, GBT), i32),
        pltpu.VMEM((CQT, GBT), i32),
        pltpu.VMEM((GBT,), f32),
        pltpu.VMEM((GBT,), f32),
        pltpu.VMEM((RPT,), f32),
        pltpu.VMEM((RPT,), f32),
        pltpu.VMEM_SHARED((NPAD,), f32),
    ] + [pltpu.SemaphoreType.DMA] * 8,
)


def _row_body(srcc_hbm, dstc_hbm, ylo_hbm, yhi_hbm, out_hbm,
              isq0, isq1, idq0, idq1, st0, st1, zrow_v, a_sh,
              sa0, sa1, sb0, sb1, sg0, sg1, ss0, ss1):
    cid = lax.axis_index("c")
    sid = lax.axis_index("s")
    ZR = 98
    isq = (isq0, isq1)
    idq = (idq0, idq1)
    st = (st0, st1)
    sa = (sa0, sa1)
    sb = (sb0, sb1)
    sg = (sg0, sg1)
    ss = (ss0, ss1)

    def zinit(i, _):
        zrow_v[i, pl.ds(0, 2 * L)] = jnp.zeros((2 * L,), bf16)
        return 0
    lax.fori_loop(0, ZR, zinit, 0)
    for k in range(RPT // ZR):
        pltpu.sync_copy(zrow_v, a_sh.at[pl.ds(sid * RPT + k * ZR, ZR)])
    plsc.subcore_barrier()

    def gather(idxref, stref, sem):
        @pl.when(cid == 0)
        def _():
            pltpu.async_copy(ylo_hbm.at[idxref], stref, sem)

        @pl.when(cid == 1)
        def _():
            pltpu.async_copy(yhi_hbm.at[idxref], stref, sem)
        return pltpu.make_async_copy(ylo_hbm.at[idxref], stref, sem)

    pltpu.async_copy(srcc_hbm.at[sid * NCHR], isq[0], sa[0])
    pltpu.async_copy(dstc_hbm.at[sid * NCHR], idq[0], sb[0])

    def do_chunk(c, cs):
        @pl.when(c + 1 < NCHR)
        def _():
            pltpu.async_copy(srcc_hbm.at[sid * NCHR + c + 1],
                             isq[1 - cs], sa[1 - cs])
            pltpu.async_copy(dstc_hbm.at[sid * NCHR + c + 1],
                             idq[1 - cs], sb[1 - cs])
        pltpu.make_async_copy(srcc_hbm.at[sid * NCHR], isq[cs], sa[cs]).wait()
        pltpu.make_async_copy(dstc_hbm.at[sid * NCHR], idq[cs], sb[cs]).wait()
        gd = [None, None]
        sd = [None, None]
        for k in range(CQR):
            slot = k % 2
            if k >= 2:
                sd[slot].wait()
            gd[slot] = gather(isq[cs].at[k], st[slot], sg[slot])
            if k >= 1:
                gd[1 - slot].wait()
                sd[1 - slot] = pltpu.async_copy(
                    st[1 - slot], a_sh.at[idq[cs].at[k - 1]], ss[1 - slot],
                    add=True)
        last = (CQR - 1) % 2
        gd[last].wait()
        sd[last] = pltpu.async_copy(
            st[last], a_sh.at[idq[cs].at[CQR - 1]], ss[last], add=True)
        sd[0].wait()
        sd[1].wait()

    def pair(i, _):
        do_chunk(2 * i, 0)
        do_chunk(2 * i + 1, 1)
        return 0
    lax.fori_loop(0, NCHR // 2, pair, 0)
    plsc.subcore_barrier()
    for k in range(RPT // ZR):
        pltpu.sync_copy(a_sh.at[pl.ds(sid * RPT + k * ZR, ZR)], zrow_v)
        pltpu.sync_copy(zrow_v,
                        out_hbm.at[pl.ds(cid * NPAD + sid * RPT + k * ZR, ZR)])


_row_call = pl.kernel(
    _row_body,
    out_type=jax.ShapeDtypeStruct((NC * NPAD, HID // 2), bf16),
    mesh=_mesh,
    compiler_params=pltpu.CompilerParams(use_tc_tiling_on_sc=False),
    scratch_types=[
        pltpu.VMEM((CQR, GBR), i32),
        pltpu.VMEM((CQR, GBR), i32),
        pltpu.VMEM((CQR, GBR), i32),
        pltpu.VMEM((CQR, GBR), i32),
        pltpu.VMEM((GBR, HID // 2), bf16),
        pltpu.VMEM((GBR, HID // 2), bf16),
        pltpu.VMEM((98, HID // 2), bf16),
        pltpu.VMEM_SHARED((NPAD, HID // 2), bf16),
    ] + [pltpu.SemaphoreType.DMA] * 8,
)


def _tc2_body(t_ref, d_ref, x_ref, s_ref):
    d = d_ref[...]
    s_ref[...] = d * (t_ref[0] + t_ref[1] + d * x_ref[...])


_tc2 = pl.pallas_call(
    _tc2_body,
    grid=(NROW // 8,),
    in_specs=[
        pl.BlockSpec((NC, 8, NLANE), lambda i: (0, i, 0)),
        pl.BlockSpec((8, NLANE), lambda i: (i, 0)),
        pl.BlockSpec((8, NLANE), lambda i: (i, 0)),
    ],
    out_specs=pl.BlockSpec((8, NLANE), lambda i: (i, 0)),
    out_shape=jax.ShapeDtypeStruct((NROW, NLANE), f32),
)

RB = 1024


def _tc3_body(t_ref, d_ref, x_ref, w1_ref, b1_ref, pa_ref, w2_ref,
              z_ref, ylo_ref, yhi_ref):
    d = d_ref[...]
    s = d * (t_ref[0] + t_ref[1] + d * x_ref[...])
    h = s * w1_ref[...] + b1_ref[...]
    a = pa_ref[0, 0]
    h = jnp.where(h >= 0, h, a * h)
    z = jnp.dot(h, w2_ref[...], precision=lax.Precision.HIGHEST,
                preferred_element_type=f32)
    y = (d * z).astype(bf16)
    z_ref[...] = z
    ylo_ref[...] = y[:, :HID // 2]
    yhi_ref[...] = y[:, HID // 2:]


_tc3 = pl.pallas_call(
    _tc3_body,
    grid=(NPAD // RB,),
    in_specs=[
        pl.BlockSpec((NC, RB, 1), lambda i: (0, i, 0)),
        pl.BlockSpec((RB, 1), lambda i: (i, 0)),
        pl.BlockSpec((RB, 1), lambda i: (i, 0)),
        pl.BlockSpec((1, HID), lambda i: (0, 0)),
        pl.BlockSpec((1, HID), lambda i: (0, 0)),
        pl.BlockSpec(memory_space=pltpu.SMEM),
        pl.BlockSpec((HID, HID), lambda i: (0, 0)),
    ],
    out_specs=[
        pl.BlockSpec((RB, HID), lambda i: (i, 0)),
        pl.BlockSpec((RB, HID // 2), lambda i: (i, 0)),
        pl.BlockSpec((RB, HID // 2), lambda i: (i, 0)),
    ],
    out_shape=[
        jax.ShapeDtypeStruct((NPAD, HID), f32),
        jax.ShapeDtypeStruct((NPAD, HID // 2), bf16),
        jax.ShapeDtypeStruct((NPAD, HID // 2), bf16),
    ],
)


def _tc4_body(alo_ref, ahi_ref, z_ref, d_ref, b2_ref, out_ref):
    d = d_ref[...]
    z = z_ref[...]
    b2 = b2_ref[...]
    alo = alo_ref[...].astype(f32)
    ahi = ahi_ref[...].astype(f32)
    lo = d * (alo + d * z[:, :HID // 2]) + b2[:, :HID // 2]
    hi = d * (ahi + d * z[:, HID // 2:]) + b2[:, HID // 2:]
    out_ref[...] = jnp.concatenate([lo, hi], axis=1)


_tc4 = pl.pallas_call(
    _tc4_body,
    grid=(NPAD // RB,),
    in_specs=[
        pl.BlockSpec((RB, HID // 2), lambda i: (i, 0)),
        pl.BlockSpec((RB, HID // 2), lambda i: (i, 0)),
        pl.BlockSpec((RB, HID), lambda i: (i, 0)),
        pl.BlockSpec((RB, 1), lambda i: (i, 0)),
        pl.BlockSpec((1, HID), lambda i: (0, 0)),
    ],
    out_specs=pl.BlockSpec((RB, HID), lambda i: (i, 0)),
    out_shape=jax.ShapeDtypeStruct((NPAD, HID), f32),
)


def kernel(data_x, data_adj, W1, b1, prelu_a, W2, b2):
    x = data_x[:, 0].astype(f32)
    xp = jnp.pad(x, (0, NPAD - N))
    src = data_adj[0].astype(i32)
    dst = data_adj[1].astype(i32)
    src2 = jnp.pad(src, (0, E2 - E))
    dst2 = jnp.pad(dst, (0, E2 - E), constant_values=NPAD - 1)
    srct = src2.reshape(E2 // (CQT * GBT), CQT, GBT)
    dstt = dst2.reshape(E2 // (CQT * GBT), CQT, GBT)
    srcc = src2.reshape(E2 // (CQR * GBR), CQR, GBR)
    dstc = dst2.reshape(E2 // (CQR * GBR), CQR, GBR)

    d_flat, _u_stage, tpart = _scal_call(srct, dstt, xp)

    t_col = tpart.reshape(NC, NPAD, 1)
    x_col = xp.reshape(NPAD, 1)
    d_col = d_flat.reshape(NPAD, 1)
    z, ylo, yhi = _tc3(t_col, d_col, x_col, W1.reshape(1, HID).astype(f32),
                       b1.reshape(1, HID).astype(f32),
                       prelu_a.reshape(1, 1).astype(f32), W2.astype(f32))

    apart = _row_call(srcc, dstc, ylo, yhi)
    outp = _tc4(apart[:NPAD], apart[NPAD:], z, d_col,
                b2.reshape(1, HID).astype(f32))
    return outp[:N]
